# baseline (speedup 1.0000x reference)
import jax
import jax.numpy as jnp
from jax import lax
from jax.experimental import pallas as pl
from jax.experimental.pallas import tpu as pltpu
from jax.experimental.pallas import tpu_sc as plsc

N_HEADS = 12
D_EMB = 64
P = 200
N_BATCH = 2
LANES = 16
TAB_STRIDE = D_EMB + 1


def _sc_body(idx_hbm, tab_hbm, out_hbm, idx_all, tabv, tr0, tr1, tr2, tr3,
             isem, wsem0, wsem1, wsem2, wsem3):
    info = plsc.get_sparse_core_info()
    nc, ns = info.num_cores, info.num_subcores
    n_workers = nc * ns
    wid = lax.axis_index("s") * nc + lax.axis_index("c")

    n_tasks = N_BATCH * P
    full_k = n_tasks // n_workers
    extra = n_tasks % n_workers
    max_k = full_k + 1

    pltpu.sync_copy(tab_hbm, tabv)

    idx_hs = []
    for k in range(max_k):
        t = wid + k * n_workers
        t = jnp.minimum(t, n_tasks - 1)
        idx_hs.append(pltpu.async_copy(
            idx_hbm.at[pl.ds(t * P, P)], idx_all.at[pl.ds(k * P, P)], isem))

    n_jg = P // LANES + 1

    def build_block(t, k, tr_v):
        bb = t // P
        i = t % P
        idx_hs[k].wait()

        def jg_body(jg, _):
            j0 = jnp.minimum(jg * LANES, P - LANES)
            w = idx_all[pl.ds(k * P + j0, LANES)] * TAB_STRIDE

            def d_body(dq, _):
                d = dq * 4
                for c in range(4):
                    tr_v[d + c, pl.ds(j0, LANES)] = plsc.load_gather(tabv, [w + (d + c)])
                return 0

            lax.fori_loop(0, D_EMB // 4, d_body, 0)
            return 0

        lax.fori_loop(0, n_jg, jg_body, 0)
        return bb, i

    def fire_writes(bb, i, tr_v, wsem):
        hs = []
        for k in range(N_HEADS):
            g = 2 * k + bb
            b_out = g // N_HEADS
            h_out = g % N_HEADS
            hs.append(pltpu.async_copy(tr_v, out_hbm.at[b_out, h_out, i], wsem))
        return hs

    nbuf = 4
    bufs = ((tr0, wsem0), (tr1, wsem1), (tr2, wsem2), (tr3, wsem3))
    pending = [None] * nbuf
    for k in range(full_k):
        tr_v, wsem = bufs[k % nbuf]
        if pending[k % nbuf] is not None:
            for h in pending[k % nbuf]:
                h.wait()
        t = wid + k * n_workers
        bb, i = build_block(t, k, tr_v)
        pending[k % nbuf] = fire_writes(bb, i, tr_v, wsem)

    @pl.when(wid < extra)
    def _():
        tr_v, wsem = bufs[full_k % nbuf]
        for h in pending[full_k % nbuf]:
            h.wait()
        t = wid + full_k * n_workers
        bb, i = build_block(t, full_k, tr_v)
        for h in fire_writes(bb, i, tr_v, wsem):
            h.wait()

    @pl.when(wid >= extra)
    def _():
        for h in pending[full_k % nbuf]:
            h.wait()
        idx_hs[full_k].wait()

    for m in range(1, nbuf):
        for h in pending[(full_k + m) % nbuf]:
            h.wait()


def kernel(src_seq, structure_emb):
    batch, num_posts, _ = src_seq.shape
    idx = src_seq.reshape(-1).astype(jnp.int32)
    tab_pad = jnp.pad(structure_emb.astype(jnp.float32), ((0, 0), (0, 1)))
    tab_flat = tab_pad.reshape(-1)

    mesh = plsc.VectorSubcoreMesh(core_axis_name="c", subcore_axis_name="s")
    f = pl.kernel(
        _sc_body,
        out_type=jax.ShapeDtypeStruct((batch, N_HEADS, num_posts, D_EMB, num_posts),
                                      jnp.float32),
        mesh=mesh,
        scratch_types=[
            pltpu.VMEM(((batch * num_posts // 32 + 1) * num_posts,), jnp.int32),
            pltpu.VMEM((tab_flat.shape[0],), jnp.float32),
            pltpu.VMEM((D_EMB, P), jnp.float32),
            pltpu.VMEM((D_EMB, P), jnp.float32),
            pltpu.VMEM((D_EMB, P), jnp.float32),
            pltpu.VMEM((D_EMB, P), jnp.float32),
            pltpu.SemaphoreType.DMA,
            pltpu.SemaphoreType.DMA,
            pltpu.SemaphoreType.DMA,
            pltpu.SemaphoreType.DMA,
            pltpu.SemaphoreType.DMA,
        ],
        compiler_params=pltpu.CompilerParams(use_tc_tiling_on_sc=True,
                                             needs_layout_passes=False),
    )
    out = f(idx, tab_flat)
    return out.transpose(0, 1, 2, 4, 3)


# speedup vs baseline: 1.0235x; 1.0235x over previous
import jax
import jax.numpy as jnp
from jax import lax
from jax.experimental import pallas as pl
from jax.experimental.pallas import tpu as pltpu
from jax.experimental.pallas import tpu_sc as plsc

N_HEADS = 12
D_EMB = 64
P = 200
N_BATCH = 2
LANES = 16
TAB_STRIDE = D_EMB + 1


def _sc_body(idx_hbm, tab_hbm, out_hbm, idx_all, tabv, tr0, tr1,
             isem, wsem0, wsem1):
    info = plsc.get_sparse_core_info()
    nc, ns = info.num_cores, info.num_subcores
    n_workers = nc * ns
    wid = lax.axis_index("s") * nc + lax.axis_index("c")

    n_tasks = N_BATCH * P
    full_k = n_tasks // n_workers
    extra = n_tasks % n_workers
    max_k = full_k + 1

    pltpu.sync_copy(tab_hbm, tabv)

    idx_hs = []
    for k in range(max_k):
        if k == full_k:
            t = (wid % extra) + k * n_workers
        else:
            t = wid + k * n_workers
        idx_hs.append(pltpu.async_copy(
            idx_hbm.at[pl.ds(t * P, P)], idx_all.at[pl.ds(k * P, P)], isem))

    n_jg = P // LANES + 1

    def build_block(t, k, tr_v):
        bb = t // P
        i = t % P
        idx_hs[k].wait()

        def jg_body(jg, _):
            j0 = jnp.minimum(jg * LANES, P - LANES)
            w = idx_all[pl.ds(k * P + j0, LANES)] * TAB_STRIDE

            def d_body(dq, _):
                d = dq * 4
                for c in range(4):
                    tr_v[d + c, pl.ds(j0, LANES)] = plsc.load_gather(tabv, [w + (d + c)])
                return 0

            lax.fori_loop(0, D_EMB // 4, d_body, 0)
            return 0

        lax.fori_loop(0, n_jg, jg_body, 0)
        return bb, i

    def fire_writes(bb, i, tr_v, wsem):
        hs = []
        for k in range(N_HEADS):
            g = 2 * k + bb
            b_out = g // N_HEADS
            h_out = g % N_HEADS
            hs.append(pltpu.async_copy(tr_v, out_hbm.at[b_out, h_out, i], wsem))
        return hs

    nbuf = 2
    bufs = ((tr0, wsem0), (tr1, wsem1))
    pending = [None] * nbuf
    for k in range(full_k):
        tr_v, wsem = bufs[k % nbuf]
        if pending[k % nbuf] is not None:
            for h in pending[k % nbuf]:
                h.wait()
        t = wid + k * n_workers
        bb, i = build_block(t, k, tr_v)
        pending[k % nbuf] = fire_writes(bb, i, tr_v, wsem)

    t13 = (wid % extra) + full_k * n_workers
    bb13 = t13 // P
    i13 = t13 % P
    j_parts = ([jg * LANES for jg in range(128 // LANES)],
               [128 + jg * LANES for jg in range(4)] + [P - LANES])
    part_j0 = (0, 128)
    part_w = (128, P - 128)

    def build_part(k, tr_v, jstarts):
        idx_hs[k].wait()
        for j0 in jstarts:
            w = idx_all[pl.ds(k * P + j0, LANES)] * TAB_STRIDE

            def d_body(dq, _):
                d = dq * 4
                for c in range(4):
                    tr_v[d + c, pl.ds(j0, LANES)] = plsc.load_gather(tabv, [w + (d + c)])
                return 0

            lax.fori_loop(0, D_EMB // 4, d_body, 0)

    for part in range(2):
        @pl.when(wid // extra == part)
        def _(part=part):
            tr_v, wsem = bufs[full_k % nbuf]
            for h in pending[full_k % nbuf]:
                h.wait()
            build_part(full_k, tr_v, j_parts[part])
            j0p, wp = part_j0[part], part_w[part]
            hs = []
            for k in range(N_HEADS):
                g = 2 * k + bb13
                hs.append(pltpu.async_copy(
                    tr_v.at[:, pl.ds(j0p, wp)],
                    out_hbm.at[g // N_HEADS, g % N_HEADS, i13, :, pl.ds(j0p, wp)],
                    wsem))
            for h in hs:
                h.wait()

    for m in range(1, nbuf):
        for h in pending[(full_k + m) % nbuf]:
            h.wait()


def kernel(src_seq, structure_emb):
    batch, num_posts, _ = src_seq.shape
    idx = src_seq.reshape(-1).astype(jnp.int32)
    tab_pad = jnp.pad(structure_emb.astype(jnp.float32), ((0, 0), (0, 1)))
    tab_flat = tab_pad.reshape(-1)

    mesh = plsc.VectorSubcoreMesh(core_axis_name="c", subcore_axis_name="s")
    f = pl.kernel(
        _sc_body,
        out_type=jax.ShapeDtypeStruct((batch, N_HEADS, num_posts, D_EMB, num_posts),
                                      jnp.float32),
        mesh=mesh,
        scratch_types=[
            pltpu.VMEM(((batch * num_posts // 32 + 1) * num_posts,), jnp.int32),
            pltpu.VMEM((tab_flat.shape[0],), jnp.float32),
            pltpu.VMEM((D_EMB, P), jnp.float32),
            pltpu.VMEM((D_EMB, P), jnp.float32),
            pltpu.SemaphoreType.DMA,
            pltpu.SemaphoreType.DMA,
            pltpu.SemaphoreType.DMA,
        ],
        compiler_params=pltpu.CompilerParams(use_tc_tiling_on_sc=True,
                                             needs_layout_passes=False),
    )
    out = f(idx, tab_flat)
    return out.transpose(0, 1, 2, 4, 3)
